# baseline (speedup 1.0000x reference)
import functools

import jax
import jax.numpy as jnp
from jax import lax
from jax.experimental import pallas as pl
from jax.experimental.pallas import tpu as pltpu
from jax.experimental.pallas import tpu_sc as plsc

_T, _N, _C, _DO, _DI = 32, 16, 2, 512, 64
_NW = 32
_TI = 8
_TO = 128
_NCHUNK = (_DI // _TI) * (_DO // _TO)
_OT = _DO // _TO
_G = (_TI * _TO) // 16

_mesh = plsc.VectorSubcoreMesh(core_axis_name="c", subcore_axis_name="s")


def _sigmoid(v):
    return 1.0 / (1.0 + jnp.exp(-v))


@functools.partial(
    pl.kernel,
    mesh=_mesh,
    out_type=jax.ShapeDtypeStruct((_T, _N, _C, _DI, _DO), jnp.float32),
    scratch_types=[
        pltpu.VMEM((_T, _TI, _TO), jnp.float32),
        pltpu.VMEM((_T, _TI, _TO), jnp.float32),
        pltpu.VMEM((_TI, _TO), jnp.float32),
        pltpu.SemaphoreType.DMA,
        pltpu.SemaphoreType.DMA,
    ],
    compiler_params=pltpu.CompilerParams(
        use_tc_tiling_on_sc=True, needs_layout_passes=False
    ),
)
def _delay_sc(x_hbm, br0_hbm, out_hbm, in_sp, out_sp, br_sp, sem_in, sem_out):
    wid = lax.axis_index("s") * 2 + lax.axis_index("c")
    n0 = wid // _C
    c0 = wid % _C

    def chunk_body(ci, carry):
        i0 = (ci // _OT) * _TI
        ob = (ci % _OT) * _TO

        copies = [
            pltpu.make_async_copy(
                x_hbm.at[t, n0, c0, pl.ds(i0, _TI), pl.ds(ob, _TO)],
                in_sp.at[t],
                sem_in,
            )
            for t in range(_T)
        ]
        cbr = pltpu.make_async_copy(
            br0_hbm.at[wid, pl.ds(i0, _TI), pl.ds(ob, _TO)], br_sp, sem_in
        )
        for cp in copies:
            cp.start()
        cbr.start()
        for cp in copies:
            cp.wait()
        cbr.wait()

        def group_body(j, c2):
            il = j >> 3
            o0 = (j & 7) << 4
            ovec = o0 + lax.iota(jnp.int32, 16)
            ivec = jnp.full((16,), il, jnp.int32)

            vals = []
            idxs = []
            for blk in range(4):
                sv = []
                iv = []
                for tt in range(8):
                    t = blk * 8 + tt
                    st = _sigmoid(in_sp[t, il, pl.ds(o0, 16)])
                    in_sp[t, il, pl.ds(o0, 16)] = st
                    sv.append(st)
                    iv.append(t)
                lvl = [
                    (sv[k], sv[k + 1], iv[k], iv[k + 1]) for k in (0, 2, 4, 6)
                ]
                sv2 = []
                iv2 = []
                for a, b, ia, ib in lvl:
                    gt = b > a
                    sv2.append(jnp.where(gt, b, a))
                    iv2.append(jnp.where(gt, ib, ia))
                gt = sv2[1] > sv2[0]
                sL = jnp.where(gt, sv2[1], sv2[0])
                iL = jnp.where(gt, iv2[1], iv2[0])
                gt = sv2[3] > sv2[2]
                sR = jnp.where(gt, sv2[3], sv2[2])
                iR = jnp.where(gt, iv2[3], iv2[2])
                gt = sR > sL
                vals.append(jnp.where(gt, sR, sL))
                idxs.append(jnp.where(gt, iR, iL))
            gt = vals[1] > vals[0]
            sA = jnp.where(gt, vals[1], vals[0])
            iA = jnp.where(gt, idxs[1], idxs[0])
            gt = vals[3] > vals[2]
            sB = jnp.where(gt, vals[3], vals[2])
            iB = jnp.where(gt, idxs[3], idxs[2])
            gt = sB > sA
            am = jnp.where(gt, iB, iA)

            brf = jnp.minimum(
                br_sp[il, pl.ds(o0, 16)], (31 - am).astype(jnp.float32)
            )
            br = brf.astype(jnp.int32)
            for t in range(_T):
                r = (t - br) & 31
                out_sp[t, il, pl.ds(o0, 16)] = plsc.load_gather(
                    in_sp, [r, ivec, ovec]
                )
            return c2

        lax.fori_loop(0, _G, group_body, 0, unroll=2)

        ocopies = [
            pltpu.make_async_copy(
                out_sp.at[t],
                out_hbm.at[t, n0, c0, pl.ds(i0, _TI), pl.ds(ob, _TO)],
                sem_out,
            )
            for t in range(_T)
        ]
        for cp in ocopies:
            cp.start()
        for cp in ocopies:
            cp.wait()
        return carry

    lax.fori_loop(0, _NCHUNK, chunk_body, 0, unroll=False)


def kernel(input, delay):
    bd = jnp.broadcast_to(delay[None, None, :, :], (_N, _C, _DO, _DI))
    bf = jnp.floor(bd)
    bern = jax.random.bernoulli(jax.random.key(1), bd - bf)
    br0 = jnp.where(bern, bf + 1.0, bf)
    br0_t = jnp.swapaxes(br0, 2, 3).reshape(_N * _C, _DI, _DO)
    x_t = jnp.swapaxes(input, 3, 4)
    out_t = _delay_sc(x_t, br0_t)
    return jnp.swapaxes(out_t, 3, 4)


# speedup vs baseline: 1.2466x; 1.2466x over previous
import functools

import jax
import jax.numpy as jnp
from jax import lax
from jax.experimental import pallas as pl
from jax.experimental.pallas import tpu as pltpu
from jax.experimental.pallas import tpu_sc as plsc

_T, _N, _C, _DO, _DI = 32, 16, 2, 512, 64
_NW = 32
_TI = 8
_TO = 128
_NCHUNK = (_DI // _TI) * (_DO // _TO)
_OT = _DO // _TO
_G = (_TI * _TO) // 16

_mesh = plsc.VectorSubcoreMesh(core_axis_name="c", subcore_axis_name="s")


def _sigmoid(v):
    return 1.0 / (1.0 + jnp.exp(-v))


@functools.partial(
    pl.kernel,
    mesh=_mesh,
    out_type=jax.ShapeDtypeStruct((_T, _N, _C, _DI, _DO), jnp.float32),
    scratch_types=[
        pltpu.VMEM((_T, _TI, _TO), jnp.float32),
        pltpu.VMEM((_T, _TI, _TO), jnp.float32),
        pltpu.VMEM((_TI, _TO), jnp.float32),
        pltpu.SemaphoreType.DMA,
        pltpu.SemaphoreType.DMA,
    ],
    compiler_params=pltpu.CompilerParams(
        use_tc_tiling_on_sc=True, needs_layout_passes=False
    ),
)
def _delay_sc(x_hbm, br0_hbm, out_hbm, in_sp, out_sp, br_sp, sem_in, sem_out):
    wid = lax.axis_index("s") * 2 + lax.axis_index("c")
    n0 = wid // _C
    c0 = wid % _C

    def chunk_body(ci, carry):
        i0 = (ci // _OT) * _TI
        ob = (ci % _OT) * _TO

        copies = [
            pltpu.make_async_copy(
                x_hbm.at[t, n0, c0, pl.ds(i0, _TI), pl.ds(ob, _TO)],
                in_sp.at[t],
                sem_in,
            )
            for t in range(_T)
        ]
        cbr = pltpu.make_async_copy(
            br0_hbm.at[wid, pl.ds(i0, _TI), pl.ds(ob, _TO)], br_sp, sem_in
        )
        for cp in copies:
            cp.start()
        cbr.start()
        for cp in copies:
            cp.wait()
        cbr.wait()

        def group_body(j, c2):
            il = j >> 3
            o0 = (j & 7) << 4
            ovec = o0 + lax.iota(jnp.int32, 16)
            ivec = jnp.full((16,), il, jnp.int32)

            srows = []
            vals = []
            idxs = []
            for blk in range(4):
                sv = []
                iv = []
                for tt in range(8):
                    t = blk * 8 + tt
                    st = _sigmoid(in_sp[t, il, pl.ds(o0, 16)])
                    srows.append(st)
                    sv.append(st)
                    iv.append(t)
                lvl = [
                    (sv[k], sv[k + 1], iv[k], iv[k + 1]) for k in (0, 2, 4, 6)
                ]
                sv2 = []
                iv2 = []
                for a, b, ia, ib in lvl:
                    gt = b > a
                    sv2.append(jnp.where(gt, b, a))
                    iv2.append(jnp.where(gt, ib, ia))
                gt = sv2[1] > sv2[0]
                sL = jnp.where(gt, sv2[1], sv2[0])
                iL = jnp.where(gt, iv2[1], iv2[0])
                gt = sv2[3] > sv2[2]
                sR = jnp.where(gt, sv2[3], sv2[2])
                iR = jnp.where(gt, iv2[3], iv2[2])
                gt = sR > sL
                vals.append(jnp.where(gt, sR, sL))
                idxs.append(jnp.where(gt, iR, iL))
            gt = vals[1] > vals[0]
            sA = jnp.where(gt, vals[1], vals[0])
            iA = jnp.where(gt, idxs[1], idxs[0])
            gt = vals[3] > vals[2]
            sB = jnp.where(gt, vals[3], vals[2])
            iB = jnp.where(gt, idxs[3], idxs[2])
            gt = sB > sA
            am = jnp.where(gt, iB, iA)

            brf = jnp.minimum(
                br_sp[il, pl.ds(o0, 16)], (31 - am).astype(jnp.float32)
            )
            br = brf.astype(jnp.int32)
            for t in range(_T):
                d = (t + br) & 31
                plsc.store_scatter(out_sp, [d, ivec, ovec], srows[t])
            return c2

        lax.fori_loop(0, _G, group_body, 0, unroll=2)

        ocopies = [
            pltpu.make_async_copy(
                out_sp.at[t],
                out_hbm.at[t, n0, c0, pl.ds(i0, _TI), pl.ds(ob, _TO)],
                sem_out,
            )
            for t in range(_T)
        ]
        for cp in ocopies:
            cp.start()
        for cp in ocopies:
            cp.wait()
        return carry

    lax.fori_loop(0, _NCHUNK, chunk_body, 0, unroll=False)


def kernel(input, delay):
    bd = jnp.broadcast_to(delay[None, None, :, :], (_N, _C, _DO, _DI))
    bf = jnp.floor(bd)
    bern = jax.random.bernoulli(jax.random.key(1), bd - bf)
    br0 = jnp.where(bern, bf + 1.0, bf)
    br0_t = jnp.swapaxes(br0, 2, 3).reshape(_N * _C, _DI, _DO)
    x_t = jnp.swapaxes(input, 3, 4)
    out_t = _delay_sc(x_t, br0_t)
    return jnp.swapaxes(out_t, 3, 4)


# speedup vs baseline: 1.2774x; 1.0247x over previous
import functools

import jax
import jax.numpy as jnp
from jax import lax
from jax.experimental import pallas as pl
from jax.experimental.pallas import tpu as pltpu
from jax.experimental.pallas import tpu_sc as plsc

_T, _N, _C, _DO, _DI = 32, 16, 2, 512, 64
_NW = 32
_TI = 8
_TO = 128
_NCHUNK = (_DI // _TI) * (_DO // _TO)
_OT = _DO // _TO
_G = (_TI * _TO) // 16

_mesh = plsc.VectorSubcoreMesh(core_axis_name="c", subcore_axis_name="s")


def _sigmoid(v):
    return 1.0 / (1.0 + jnp.exp(-v))


@functools.partial(
    pl.kernel,
    mesh=_mesh,
    out_type=jax.ShapeDtypeStruct((_T, _N, _C, _DI, _DO), jnp.float32),
    scratch_types=[
        pltpu.VMEM((_T, _TI, _TO), jnp.float32),
        pltpu.VMEM((_T, _TI, _TO), jnp.float32),
        pltpu.VMEM((_TI, _TO), jnp.float32),
        pltpu.SemaphoreType.DMA,
        pltpu.SemaphoreType.DMA,
    ],
    compiler_params=pltpu.CompilerParams(
        use_tc_tiling_on_sc=True, needs_layout_passes=False
    ),
)
def _delay_sc(x_hbm, br0_hbm, out_hbm, in_sp, out_sp, br_sp, sem_in, sem_out):
    wid = lax.axis_index("s") * 2 + lax.axis_index("c")
    n0 = wid // _C
    c0 = wid % _C

    def chunk_body(ci, carry):
        i0 = (ci // _OT) * _TI
        ob = (ci % _OT) * _TO

        copies = [
            pltpu.make_async_copy(
                x_hbm.at[t, n0, c0, pl.ds(i0, _TI), pl.ds(ob, _TO)],
                in_sp.at[t],
                sem_in,
            )
            for t in range(_T)
        ]
        cbr = pltpu.make_async_copy(
            br0_hbm.at[wid, pl.ds(i0, _TI), pl.ds(ob, _TO)], br_sp, sem_in
        )
        for cp in copies:
            cp.start()
        cbr.start()
        for cp in copies:
            cp.wait()
        cbr.wait()

        @plsc.parallel_loop(0, _G, 1, unroll=2)
        def group_body(j):
            il = j >> 3
            o0 = (j & 7) << 4
            ovec = o0 + lax.iota(jnp.int32, 16)
            ivec = jnp.full((16,), il, jnp.int32)

            srows = []
            vals = []
            idxs = []
            for blk in range(4):
                sv = []
                iv = []
                for tt in range(8):
                    t = blk * 8 + tt
                    st = _sigmoid(in_sp[t, il, pl.ds(o0, 16)])
                    srows.append(st)
                    sv.append(st)
                    iv.append(t)
                lvl = [
                    (sv[k], sv[k + 1], iv[k], iv[k + 1]) for k in (0, 2, 4, 6)
                ]
                sv2 = []
                iv2 = []
                for a, b, ia, ib in lvl:
                    gt = b > a
                    sv2.append(jnp.where(gt, b, a))
                    iv2.append(jnp.where(gt, ib, ia))
                gt = sv2[1] > sv2[0]
                sL = jnp.where(gt, sv2[1], sv2[0])
                iL = jnp.where(gt, iv2[1], iv2[0])
                gt = sv2[3] > sv2[2]
                sR = jnp.where(gt, sv2[3], sv2[2])
                iR = jnp.where(gt, iv2[3], iv2[2])
                gt = sR > sL
                vals.append(jnp.where(gt, sR, sL))
                idxs.append(jnp.where(gt, iR, iL))
            gt = vals[1] > vals[0]
            sA = jnp.where(gt, vals[1], vals[0])
            iA = jnp.where(gt, idxs[1], idxs[0])
            gt = vals[3] > vals[2]
            sB = jnp.where(gt, vals[3], vals[2])
            iB = jnp.where(gt, idxs[3], idxs[2])
            gt = sB > sA
            am = jnp.where(gt, iB, iA)

            brf = jnp.minimum(
                br_sp[il, pl.ds(o0, 16)], (31 - am).astype(jnp.float32)
            )
            br = brf.astype(jnp.int32)
            for t in range(_T):
                d = (t + br) & 31
                plsc.store_scatter(out_sp, [d, ivec, ovec], srows[t])

        ocopies = [
            pltpu.make_async_copy(
                out_sp.at[t],
                out_hbm.at[t, n0, c0, pl.ds(i0, _TI), pl.ds(ob, _TO)],
                sem_out,
            )
            for t in range(_T)
        ]
        for cp in ocopies:
            cp.start()
        for cp in ocopies:
            cp.wait()
        return carry

    lax.fori_loop(0, _NCHUNK, chunk_body, 0, unroll=False)


def kernel(input, delay):
    bd = jnp.broadcast_to(delay[None, None, :, :], (_N, _C, _DO, _DI))
    bf = jnp.floor(bd)
    bern = jax.random.bernoulli(jax.random.key(1), bd - bf)
    br0 = jnp.where(bern, bf + 1.0, bf)
    br0_t = jnp.swapaxes(br0, 2, 3).reshape(_N * _C, _DI, _DO)
    x_t = jnp.swapaxes(input, 3, 4)
    out_t = _delay_sc(x_t, br0_t)
    return jnp.swapaxes(out_t, 3, 4)


# speedup vs baseline: 1.9007x; 1.4879x over previous
import functools

import jax
import jax.numpy as jnp
from jax import lax
from jax.experimental import pallas as pl
from jax.experimental.pallas import tpu as pltpu
from jax.experimental.pallas import tpu_sc as plsc

_T, _N, _C, _DO, _DI = 32, 16, 2, 512, 64
_NW = 32
_TI = 8
_TO = 128
_NCHUNK = (_DI // _TI) * (_DO // _TO)
_OT = _DO // _TO
_G = (_TI * _TO) // 16

_mesh = plsc.VectorSubcoreMesh(core_axis_name="c", subcore_axis_name="s")


def _sigmoid(v):
    y = 1.0 + jnp.exp(-v)
    bits = lax.bitcast_convert_type(y, jnp.int32)
    r = lax.bitcast_convert_type(jnp.int32(0x7EF311C3) - bits, jnp.float32)
    r = r * (2.0 - y * r)
    r = r * (2.0 - y * r)
    r = r * (2.0 - y * r)
    return r


@functools.partial(
    pl.kernel,
    mesh=_mesh,
    out_type=jax.ShapeDtypeStruct((_T, _N, _C, _DI, _DO), jnp.float32),
    scratch_types=[
        pltpu.VMEM((_T, _TI, _TO), jnp.float32),
        pltpu.VMEM((_T, _TI, _TO), jnp.float32),
        pltpu.VMEM((_TI, _TO), jnp.float32),
        pltpu.SemaphoreType.DMA,
        pltpu.SemaphoreType.DMA,
    ],
    compiler_params=pltpu.CompilerParams(
        use_tc_tiling_on_sc=True, needs_layout_passes=False
    ),
)
def _delay_sc(x_hbm, br0_hbm, out_hbm, in_sp, out_sp, br_sp, sem_in, sem_out):
    wid = lax.axis_index("s") * 2 + lax.axis_index("c")
    n0 = wid // _C
    c0 = wid % _C

    def chunk_body(ci, carry):
        i0 = (ci // _OT) * _TI
        ob = (ci % _OT) * _TO

        copies = [
            pltpu.make_async_copy(
                x_hbm.at[t, n0, c0, pl.ds(i0, _TI), pl.ds(ob, _TO)],
                in_sp.at[t],
                sem_in,
            )
            for t in range(_T)
        ]
        cbr = pltpu.make_async_copy(
            br0_hbm.at[wid, pl.ds(i0, _TI), pl.ds(ob, _TO)], br_sp, sem_in
        )
        for cp in copies:
            cp.start()
        cbr.start()
        for cp in copies:
            cp.wait()
        cbr.wait()

        @plsc.parallel_loop(0, _G, 1, unroll=2)
        def group_body(j):
            il = j >> 3
            o0 = (j & 7) << 4
            ovec = o0 + lax.iota(jnp.int32, 16)
            ivec = jnp.full((16,), il, jnp.int32)

            srows = []
            vals = []
            idxs = []
            for blk in range(4):
                sv = []
                iv = []
                for tt in range(8):
                    t = blk * 8 + tt
                    st = _sigmoid(in_sp[t, il, pl.ds(o0, 16)])
                    srows.append(st)
                    sv.append(st)
                    iv.append(t)
                lvl = [
                    (sv[k], sv[k + 1], iv[k], iv[k + 1]) for k in (0, 2, 4, 6)
                ]
                sv2 = []
                iv2 = []
                for a, b, ia, ib in lvl:
                    gt = b > a
                    sv2.append(jnp.where(gt, b, a))
                    iv2.append(jnp.where(gt, ib, ia))
                gt = sv2[1] > sv2[0]
                sL = jnp.where(gt, sv2[1], sv2[0])
                iL = jnp.where(gt, iv2[1], iv2[0])
                gt = sv2[3] > sv2[2]
                sR = jnp.where(gt, sv2[3], sv2[2])
                iR = jnp.where(gt, iv2[3], iv2[2])
                gt = sR > sL
                vals.append(jnp.where(gt, sR, sL))
                idxs.append(jnp.where(gt, iR, iL))
            gt = vals[1] > vals[0]
            sA = jnp.where(gt, vals[1], vals[0])
            iA = jnp.where(gt, idxs[1], idxs[0])
            gt = vals[3] > vals[2]
            sB = jnp.where(gt, vals[3], vals[2])
            iB = jnp.where(gt, idxs[3], idxs[2])
            gt = sB > sA
            am = jnp.where(gt, iB, iA)

            brf = jnp.minimum(
                br_sp[il, pl.ds(o0, 16)], (31 - am).astype(jnp.float32)
            )
            br = brf.astype(jnp.int32)
            for t in range(_T):
                d = (t + br) & 31
                plsc.store_scatter(out_sp, [d, ivec, ovec], srows[t])

        ocopies = [
            pltpu.make_async_copy(
                out_sp.at[t],
                out_hbm.at[t, n0, c0, pl.ds(i0, _TI), pl.ds(ob, _TO)],
                sem_out,
            )
            for t in range(_T)
        ]
        for cp in ocopies:
            cp.start()
        for cp in ocopies:
            cp.wait()
        return carry

    lax.fori_loop(0, _NCHUNK, chunk_body, 0, unroll=False)


def kernel(input, delay):
    bd = jnp.broadcast_to(delay[None, None, :, :], (_N, _C, _DO, _DI))
    bf = jnp.floor(bd)
    bern = jax.random.bernoulli(jax.random.key(1), bd - bf)
    br0 = jnp.where(bern, bf + 1.0, bf)
    br0_t = jnp.swapaxes(br0, 2, 3).reshape(_N * _C, _DI, _DO)
    x_t = jnp.swapaxes(input, 3, 4)
    out_t = _delay_sc(x_t, br0_t)
    return jnp.swapaxes(out_t, 3, 4)


# speedup vs baseline: 1.9651x; 1.0339x over previous
import functools

import jax
import jax.numpy as jnp
from jax import lax
from jax.experimental import pallas as pl
from jax.experimental.pallas import tpu as pltpu
from jax.experimental.pallas import tpu_sc as plsc

_T, _N, _C, _DO, _DI = 32, 16, 2, 512, 64
_NW = 32
_TI = 8
_TO = 128
_NCHUNK = (_DI // _TI) * (_DO // _TO)
_OT = _DO // _TO
_G = (_TI * _TO) // 16

_mesh = plsc.VectorSubcoreMesh(core_axis_name="c", subcore_axis_name="s")


def _sigmoid(v):
    y = 1.0 + jnp.exp(-v)
    bits = lax.bitcast_convert_type(y, jnp.int32)
    r = lax.bitcast_convert_type(jnp.int32(0x7EF311C3) - bits, jnp.float32)
    r = r * (2.0 - y * r)
    r = r * (2.0 - y * r)
    r = r * (2.0 - y * r)
    return r


@functools.partial(
    pl.kernel,
    mesh=_mesh,
    out_type=jax.ShapeDtypeStruct((_T, _N, _C, _DI, _DO), jnp.float32),
    scratch_types=[
        pltpu.VMEM((_T, _TI, _TO), jnp.float32),
        pltpu.VMEM((_T, _TI, _TO), jnp.float32),
        pltpu.VMEM((_TI, _TO), jnp.float32),
        pltpu.SemaphoreType.DMA,
        pltpu.SemaphoreType.DMA,
    ],
    compiler_params=pltpu.CompilerParams(
        use_tc_tiling_on_sc=True, needs_layout_passes=False
    ),
)
def _delay_sc(x_hbm, br0_hbm, out_hbm, in_sp, out_sp, br_sp, sem_in, sem_out):
    wid = lax.axis_index("s") * 2 + lax.axis_index("c")
    n0 = wid // _C
    c0 = wid % _C

    def chunk_body(ci, carry):
        i0 = (ci // _OT) * _TI
        ob = (ci % _OT) * _TO

        copies = [
            pltpu.make_async_copy(
                x_hbm.at[t, n0, c0, pl.ds(i0, _TI), pl.ds(ob, _TO)],
                in_sp.at[t],
                sem_in,
            )
            for t in range(_T)
        ]
        cbr = pltpu.make_async_copy(
            br0_hbm.at[wid, pl.ds(i0, _TI), pl.ds(ob, _TO)], br_sp, sem_in
        )
        for cp in copies:
            cp.start()
        cbr.start()
        for cp in copies:
            cp.wait()
        cbr.wait()

        @plsc.parallel_loop(0, _G, 1, unroll=2)
        def group_body(j):
            il = j >> 3
            o0 = (j & 7) << 4
            ovec = o0 + lax.iota(jnp.int32, 16)
            ivec = jnp.full((16,), il, jnp.int32)

            srows = []
            vals = []
            idxs = []
            for blk in range(4):
                sv = []
                iv = []
                for tt in range(8):
                    t = blk * 8 + tt
                    xt = in_sp[t, il, pl.ds(o0, 16)]
                    srows.append(_sigmoid(xt))
                    sv.append(xt)
                    iv.append(t)
                lvl = [
                    (sv[k], sv[k + 1], iv[k], iv[k + 1]) for k in (0, 2, 4, 6)
                ]
                sv2 = []
                iv2 = []
                for a, b, ia, ib in lvl:
                    gt = b > a
                    sv2.append(jnp.where(gt, b, a))
                    iv2.append(jnp.where(gt, ib, ia))
                gt = sv2[1] > sv2[0]
                sL = jnp.where(gt, sv2[1], sv2[0])
                iL = jnp.where(gt, iv2[1], iv2[0])
                gt = sv2[3] > sv2[2]
                sR = jnp.where(gt, sv2[3], sv2[2])
                iR = jnp.where(gt, iv2[3], iv2[2])
                gt = sR > sL
                vals.append(jnp.where(gt, sR, sL))
                idxs.append(jnp.where(gt, iR, iL))
            gt = vals[1] > vals[0]
            sA = jnp.where(gt, vals[1], vals[0])
            iA = jnp.where(gt, idxs[1], idxs[0])
            gt = vals[3] > vals[2]
            sB = jnp.where(gt, vals[3], vals[2])
            iB = jnp.where(gt, idxs[3], idxs[2])
            gt = sB > sA
            am = jnp.where(gt, iB, iA)

            brf = jnp.minimum(
                br_sp[il, pl.ds(o0, 16)], (31 - am).astype(jnp.float32)
            )
            br = brf.astype(jnp.int32)
            for t in range(_T):
                d = (t + br) & 31
                plsc.store_scatter(out_sp, [d, ivec, ovec], srows[t])

        ocopies = [
            pltpu.make_async_copy(
                out_sp.at[t],
                out_hbm.at[t, n0, c0, pl.ds(i0, _TI), pl.ds(ob, _TO)],
                sem_out,
            )
            for t in range(_T)
        ]
        for cp in ocopies:
            cp.start()
        for cp in ocopies:
            cp.wait()
        return carry

    lax.fori_loop(0, _NCHUNK, chunk_body, 0, unroll=False)


def kernel(input, delay):
    bd = jnp.broadcast_to(delay[None, None, :, :], (_N, _C, _DO, _DI))
    bf = jnp.floor(bd)
    bern = jax.random.bernoulli(jax.random.key(1), bd - bf)
    br0 = jnp.where(bern, bf + 1.0, bf)
    br0_t = jnp.swapaxes(br0, 2, 3).reshape(_N * _C, _DI, _DO)
    x_t = jnp.swapaxes(input, 3, 4)
    out_t = _delay_sc(x_t, br0_t)
    return jnp.swapaxes(out_t, 3, 4)
